# baseline (speedup 1.0000x reference)
import jax
import jax.numpy as jnp
from jax.experimental import pallas as pl
from jax.experimental.pallas import tpu as pltpu

STATE_DIM = 4096
HIDDEN = 1024
MAX_PARALLEL = 32
SEQ_LEN = 2048
BATCH = 128

_NEG_INF = float("-inf")
_N1 = 8
_N2 = 2
_NQ1 = 2
_NQ2 = 4
_LOOKAHEAD = 3
_N_SMALL = 12
_N_BIG = _N1 + _N2 + _NQ1 + 1 + _NQ2


def _layernorm(x, g, b, eps=1e-5):
    mu = jnp.mean(x, axis=-1, keepdims=True)
    xc = x - mu
    var = jnp.mean(xc * xc, axis=-1, keepdims=True)
    return xc * jax.lax.rsqrt(var + eps) * g + b


def _dot_nt(a, b):
    return jax.lax.dot_general(
        a.astype(jnp.bfloat16), b.astype(jnp.bfloat16),
        (((1,), (1,)), ((), ())), preferred_element_type=jnp.float32
    )


def _fused_kernel(state_hbm, mask_hbm,
                  W1_hbm, b1_hbm, g1_hbm, be1_hbm,
                  W2_hbm, b2_hbm, g2_hbm, be2_hbm,
                  Wp1_hbm, bp1_hbm, Wp2_hbm, bp2_hbm,
                  Wq1_hbm, bq1_hbm, Wq2_hbm, bq2_hbm,
                  pos_hbm, par_ref,
                  st_buf, mask_buf, wp2_buf,
                  vb1, vg1, vbe1, vb2, vg2, vbe2, vbp1, vbp2, vbq1, vbq2,
                  w1_buf, w2_buf, wp1_buf, wq1_buf, wq2_buf,
                  h_buf, pos_buf, small_sems, big_sems, out_sems):
    vec_pairs = [(b1_hbm, vb1), (g1_hbm, vg1), (be1_hbm, vbe1),
                 (b2_hbm, vb2), (g2_hbm, vg2), (be2_hbm, vbe2),
                 (bp1_hbm, vbp1), (bp2_hbm, vbp2),
                 (bq1_hbm, vbq1), (bq2_hbm, vbq2)]
    smalls = [pltpu.make_async_copy(state_hbm, st_buf, small_sems.at[0]),
              pltpu.make_async_copy(mask_hbm, mask_buf, small_sems.at[1]),
              pltpu.make_async_copy(Wp2_hbm, wp2_buf, small_sems.at[2])]
    for i, (s, d) in enumerate(vec_pairs):
        smalls.append(pltpu.make_async_copy(s, d, small_sems.at[3 + i]))
    for c in smalls:
        c.start()

    bigs = []

    def chunks(hbm_ref, buf, n):
        rows = hbm_ref.shape[0] // n
        out = []
        for i in range(n):
            bigs.append(pltpu.make_async_copy(
                hbm_ref.at[pl.ds(i * rows, rows), :], buf.at[i],
                big_sems.at[len(bigs)]))
            out.append(len(bigs) - 1)
        return out

    i_w1 = chunks(W1_hbm, w1_buf, _N1)
    i_w2 = chunks(W2_hbm, w2_buf, _N2)
    i_wq1 = chunks(Wq1_hbm, wq1_buf, _NQ1)
    i_wp1 = chunks(Wp1_hbm, wp1_buf, 1)[0]
    i_wq2 = chunks(Wq2_hbm, wq2_buf, _NQ2)

    started = [0]

    def wait(idx):
        upto = min(idx + 1 + _LOOKAHEAD, len(bigs))
        while started[0] < upto:
            bigs[started[0]].start()
            started[0] += 1
        bigs[idx].wait()

    smalls[0].wait()
    state = st_buf[...]
    n1 = HIDDEN // _N1
    for k, idx in enumerate(i_w1):
        wait(idx)
        h_buf[:, k * n1:(k + 1) * n1] = _dot_nt(state, w1_buf[k])

    for c in smalls[1:]:
        c.wait()
    h = jnp.maximum(h_buf[...] + vb1[...], 0.0)
    h = _layernorm(h, vg1[...], vbe1[...])

    parts = []
    for k, idx in enumerate(i_w2):
        wait(idx)
        parts.append(_dot_nt(h, w2_buf[k]))
    h = jnp.maximum(jnp.concatenate(parts, axis=1) + vb2[...], 0.0)
    features = _layernorm(h, vg2[...], vbe2[...])

    mask = mask_buf[...].astype(jnp.float32)

    parts = []
    for k, idx in enumerate(i_wq1):
        wait(idx)
        parts.append(_dot_nt(features, wq1_buf[k]))
    qh = jnp.maximum(jnp.concatenate(parts, axis=1) + vbq1[...], 0.0)

    wait(i_wp1)
    ph = jnp.maximum(_dot_nt(features, wp1_buf[0]) + vbp1[...], 0.0)
    par = _dot_nt(ph, wp2_buf[...]) + vbp2[...]
    remaining = (SEQ_LEN - jnp.sum(mask, axis=-1,
                                   keepdims=True)).astype(jnp.int32)
    col = jax.lax.broadcasted_iota(jnp.int32, (BATCH, MAX_PARALLEL), 1)
    par_ref[...] = jnp.where(col >= remaining, _NEG_INF, par)

    nq2 = SEQ_LEN // _NQ2
    out_copies = []
    for k, idx in enumerate(i_wq2):
        wait(idx)
        sl = slice(k * nq2, (k + 1) * nq2)
        pos = _dot_nt(qh, wq2_buf[k]) + vbq2[:, sl]
        pos_buf[:, sl] = jnp.where(mask[:, sl] > 0, _NEG_INF, pos)
        oc = pltpu.make_async_copy(
            pos_buf.at[:, pl.ds(k * nq2, nq2)],
            pos_hbm.at[:, pl.ds(k * nq2, nq2)],
            out_sems.at[k])
        oc.start()
        out_copies.append(oc)
    for oc in out_copies:
        oc.wait()


@jax.jit
def kernel(state, generated_mask, W1, b1, g1, be1, W2, b2, g2, be2,
           Wp1, bp1, Wp2, bp2, Wq1, bq1, Wq2, bq2):
    mask8 = generated_mask.astype(jnp.int8)
    vec = lambda v: v.reshape(1, -1)
    hbm = pl.BlockSpec(memory_space=pl.ANY)
    hbm_args = (state, mask8,
                W1, vec(b1), vec(g1), vec(be1),
                W2, vec(b2), vec(g2), vec(be2),
                Wp1, vec(bp1), Wp2, vec(bp2),
                Wq1, vec(bq1), Wq2, vec(bq2))
    pos, par = pl.pallas_call(
        _fused_kernel,
        grid=(),
        in_specs=[hbm] * len(hbm_args),
        out_specs=(
            pl.BlockSpec(memory_space=pl.ANY),
            pl.BlockSpec((BATCH, MAX_PARALLEL), lambda: (0, 0)),
        ),
        out_shape=(
            jax.ShapeDtypeStruct((BATCH, SEQ_LEN), jnp.float32),
            jax.ShapeDtypeStruct((BATCH, MAX_PARALLEL), jnp.float32),
        ),
        scratch_shapes=[
            pltpu.VMEM((BATCH, STATE_DIM), jnp.float32),
            pltpu.VMEM((BATCH, SEQ_LEN), jnp.int8),
            pltpu.VMEM((MAX_PARALLEL, HIDDEN // 2), jnp.float32),
            pltpu.VMEM((1, HIDDEN), jnp.float32),
            pltpu.VMEM((1, HIDDEN), jnp.float32),
            pltpu.VMEM((1, HIDDEN), jnp.float32),
            pltpu.VMEM((1, HIDDEN), jnp.float32),
            pltpu.VMEM((1, HIDDEN), jnp.float32),
            pltpu.VMEM((1, HIDDEN), jnp.float32),
            pltpu.VMEM((1, HIDDEN // 2), jnp.float32),
            pltpu.VMEM((1, MAX_PARALLEL), jnp.float32),
            pltpu.VMEM((1, HIDDEN), jnp.float32),
            pltpu.VMEM((1, SEQ_LEN), jnp.float32),
            pltpu.VMEM((_N1, HIDDEN // _N1, STATE_DIM), jnp.float32),
            pltpu.VMEM((_N2, HIDDEN // _N2, HIDDEN), jnp.float32),
            pltpu.VMEM((1, HIDDEN // 2, HIDDEN), jnp.float32),
            pltpu.VMEM((_NQ1, HIDDEN // _NQ1, HIDDEN), jnp.float32),
            pltpu.VMEM((_NQ2, SEQ_LEN // _NQ2, HIDDEN), jnp.float32),
            pltpu.VMEM((BATCH, HIDDEN), jnp.float32),
            pltpu.VMEM((BATCH, SEQ_LEN), jnp.float32),
            pltpu.SemaphoreType.DMA((_N_SMALL + 1,)),
            pltpu.SemaphoreType.DMA((_N_BIG,)),
            pltpu.SemaphoreType.DMA((_NQ2,)),
        ],
        compiler_params=pltpu.CompilerParams(
            vmem_limit_bytes=100 * 1024 * 1024,
        ),
    )(*hbm_args)
    return (par, pos)
